# Initial kernel scaffold; baseline (speedup 1.0000x reference)
import jax
import jax.numpy as jnp
from jax.experimental import pallas as pl


def kernel(node_feat, edge_attr, edge_index, batch, W_in_node, b_in_node, W_in_edge, b_in_edge, msg1_W1, msg1_b1, msg1_W2, msg1_b2, upd1_W1, upd1_b1, upd1_W2, upd1_b2, msg2_W1, msg2_b1, msg2_W2, msg2_b2, upd2_W1, upd2_b1, upd2_W2, upd2_b2, fin_W1, fin_b1, fin_W2, fin_b2):
    raise NotImplementedError("write your pallas kernel here")


# speedup vs baseline: 14.3284x; 14.3284x over previous
import functools

import jax
import jax.numpy as jnp
from jax import lax
from jax.experimental import pallas as pl
from jax.experimental.pallas import tpu as pltpu
from jax.experimental.pallas import tpu_sc as plsc

N = 10000
E = 320000
DF = 128
DE = 16
FD = 2
HD = 64
NC = 10
K = 3

NW = 32
NB = 320
NP = NW * NB
CH = E // NW
GV = NB // 16

OCC_BASE = 1


def _wid():
    return lax.axis_index("s") * 2 + lax.axis_index("c")


@functools.lru_cache(maxsize=None)
def _sc_kernels():
    mesh = plsc.VectorSubcoreMesh(core_axis_name="c", subcore_axis_name="s")

    @functools.partial(
        pl.kernel,
        out_type=(
            jax.ShapeDtypeStruct((NW * NP,), jnp.int32),
            jax.ShapeDtypeStruct((NW * NP * K,), jnp.int32),
            jax.ShapeDtypeStruct((NW * NP * K,), jnp.int32),
        ),
        mesh=mesh,
        compiler_params=pltpu.CompilerParams(
            needs_layout_passes=False, use_tc_tiling_on_sc=False),
        scratch_types=[
            pltpu.VMEM((CH,), jnp.int32),
            pltpu.VMEM((CH,), jnp.int32),
            pltpu.VMEM((NP,), jnp.int32),
            pltpu.VMEM((NP * K,), jnp.int32),
            pltpu.VMEM((NP * K,), jnp.int32),
        ],
    )
    def sc_count(src_hbm, dst_hbm, cnt_hbm, ceid_hbm, cdst_hbm,
                 src_v, dst_v, cnt_v, ceid_v, cdst_v):
        w = _wid()
        e0 = w * CH
        pltpu.sync_copy(src_hbm.at[pl.ds(e0, CH)], src_v)
        pltpu.sync_copy(dst_hbm.at[pl.ds(e0, CH)], dst_v)

        zero16 = jnp.zeros((16,), jnp.int32)

        def zbody(i, c):
            cnt_v[pl.ds(i * 16, 16)] = zero16
            return c

        lax.fori_loop(0, NP // 16, zbody, 0)

        iota16 = lax.iota(jnp.int32, 16)

        def body(i, c):
            s16 = src_v[pl.ds(i * 16, 16)]
            d16 = dst_v[pl.ds(i * 16, 16)]
            eid = iota16 + (e0 + i * 16)
            occ, last = plsc.scan_count(s16)
            cold = plsc.load_gather(cnt_v, [s16])
            rnk = cold + occ - OCC_BASE
            plsc.store_scatter(cnt_v, [s16], rnk + 1, mask=last)
            sel = (rnk >= 0) & (rnk < K)
            slot = s16 * K + jnp.clip(rnk, 0, K - 1)
            plsc.store_scatter(ceid_v, [slot], eid, mask=sel)
            plsc.store_scatter(cdst_v, [slot], d16, mask=sel)
            return c

        lax.fori_loop(0, CH // 16, body, 0)

        pltpu.sync_copy(cnt_v, cnt_hbm.at[pl.ds(w * NP, NP)])
        pltpu.sync_copy(ceid_v, ceid_hbm.at[pl.ds(w * NP * K, NP * K)])
        pltpu.sync_copy(cdst_v, cdst_hbm.at[pl.ds(w * NP * K, NP * K)])

    @functools.partial(
        pl.kernel,
        out_type=(
            jax.ShapeDtypeStruct((K * NP,), jnp.int32),
            jax.ShapeDtypeStruct((NP,), jnp.int32),
            jax.ShapeDtypeStruct((K * NP, DE), jnp.float32),
            jax.ShapeDtypeStruct((K * NP,), jnp.float32),
            jax.ShapeDtypeStruct((K * NP,), jnp.float32),
        ),
        mesh=mesh,
        compiler_params=pltpu.CompilerParams(
            needs_layout_passes=False, use_tc_tiling_on_sc=False),
        scratch_types=[
            pltpu.VMEM((NW * NB,), jnp.int32),
            pltpu.VMEM((NW * NB * K,), jnp.int32),
            pltpu.VMEM((NW * NB * K,), jnp.int32),
            pltpu.VMEM((K * NB,), jnp.int32),
            pltpu.VMEM((K * NB,), jnp.int32),
            pltpu.VMEM((NB,), jnp.int32),
            pltpu.VMEM((128, DE), jnp.float32),
            pltpu.VMEM((128,), jnp.int32),
            pltpu.VMEM((NP,), jnp.int32),
            pltpu.VMEM((NP,), jnp.int32),
            pltpu.VMEM((K * NB,), jnp.float32),
            pltpu.VMEM((K * NB,), jnp.float32),
            pltpu.SemaphoreType.DMA,
            pltpu.SemaphoreType.DMA,
        ],
    )
    def sc_merge(cnt_hbm, ceid_hbm, cdst_hbm, eattr_hbm, na_hbm, nb_hbm,
                 nbrs_hbm, has_hbm, esel_hbm, nfa_hbm, nfb_hbm,
                 cnt_v, ceid_v, cdst_v, slots_v, nbrs_v, has_v, ebuf_v,
                 idx_v, na_v, nb_v, nfa_v, nfb_v, sem, gsem):
        w = _wid()
        n0 = w * NB
        pending = [(na_hbm, na_v), (nb_hbm, nb_v)]
        for t in range(NW):
            pending.append((cnt_hbm.at[pl.ds(t * NP + n0, NB)],
                            cnt_v.at[pl.ds(t * NB, NB)]))
            pending.append((ceid_hbm.at[pl.ds((t * NP + n0) * K, NB * K)],
                            ceid_v.at[pl.ds(t * NB * K, NB * K)]))
            pending.append((cdst_hbm.at[pl.ds((t * NP + n0) * K, NB * K)],
                            cdst_v.at[pl.ds(t * NB * K, NB * K)]))
        for i in range(0, len(pending), 8):
            cps = [pltpu.async_copy(a, b, sem) for a, b in pending[i:i + 8]]
            for cp in cps:
                cp.wait()

        zero16i = jnp.zeros((16,), jnp.int32)
        iota16 = lax.iota(jnp.int32, 16)

        def zbody(i, c):
            slots_v[pl.ds(i * 16, 16)] = zero16i
            nbrs_v[pl.ds(i * 16, 16)] = zero16i
            return c

        lax.fori_loop(0, K * GV, zbody, 0)

        def g_body(g, c):
            nidx = iota16 + g * 16

            def t_body(t, run):
                cc = cnt_v[pl.ds(t * NB + g * 16, 16)]
                cbase = t * (NB * K) + nidx * K
                for k in range(K):
                    sel = (cc > k) & (run < K - k)
                    sl = jnp.minimum(run + k, K - 1)
                    ev = plsc.load_gather(ceid_v, [cbase + k])
                    dv = plsc.load_gather(cdst_v, [cbase + k])
                    plsc.store_scatter(slots_v, [sl * NB + nidx], ev, mask=sel)
                    plsc.store_scatter(nbrs_v, [sl * NB + nidx], dv, mask=sel)
                return run + cc

            run = lax.fori_loop(0, NW, t_body, zero16i)
            has_v[pl.ds(g * 16, 16)] = (run >= K).astype(jnp.int32)
            return c

        lax.fori_loop(0, GV, g_body, 0)

        for k in range(K):
            for off, sz in ((0, 128), (128, 128), (256, 64)):
                for j in range(sz // 16):
                    idx_v[pl.ds(j * 16, 16)] = jnp.clip(
                        slots_v[pl.ds(k * NB + off + j * 16, 16)], 0, E - 1)
                src = (eattr_hbm.at[idx_v] if sz == 128
                       else eattr_hbm.at[idx_v.at[pl.ds(0, sz)]])
                pltpu.async_copy(
                    src, ebuf_v.at[pl.ds(0, sz), :], gsem).wait()
                pltpu.sync_copy(
                    ebuf_v.at[pl.ds(0, sz), :],
                    esel_hbm.at[pl.ds(k * NP + n0 + off, sz), :])

        def nf_body(i, c):
            idx = jnp.clip(nbrs_v[pl.ds(i * 16, 16)], 0, NP - 1)
            va = plsc.load_gather(na_v, [idx])
            vb = plsc.load_gather(nb_v, [idx])
            nfa_v[pl.ds(i * 16, 16)] = plsc.bitcast(va, jnp.float32)
            nfb_v[pl.ds(i * 16, 16)] = plsc.bitcast(vb, jnp.float32)
            return c

        lax.fori_loop(0, K * GV, nf_body, 0)

        for k in range(K):
            pltpu.sync_copy(nbrs_v.at[pl.ds(k * NB, NB)],
                            nbrs_hbm.at[pl.ds(k * NP + n0, NB)])
            pltpu.sync_copy(nfa_v.at[pl.ds(k * NB, NB)],
                            nfa_hbm.at[pl.ds(k * NP + n0, NB)])
            pltpu.sync_copy(nfb_v.at[pl.ds(k * NB, NB)],
                            nfb_hbm.at[pl.ds(k * NP + n0, NB)])
        pltpu.sync_copy(has_v, has_hbm.at[pl.ds(n0, NB)])

    @functools.partial(
        pl.kernel,
        out_type=(
            jax.ShapeDtypeStruct((K * NP,), jnp.float32),
            jax.ShapeDtypeStruct((K * NP,), jnp.float32),
        ),
        mesh=mesh,
        compiler_params=pltpu.CompilerParams(
            needs_layout_passes=False, use_tc_tiling_on_sc=False),
        scratch_types=[
            pltpu.VMEM((NP,), jnp.int32),
            pltpu.VMEM((NP,), jnp.int32),
            pltpu.VMEM((K * NB,), jnp.int32),
            pltpu.VMEM((K * NB,), jnp.float32),
            pltpu.VMEM((K * NB,), jnp.float32),
        ],
    )
    def sc_gather(na_hbm, nb_hbm, nbrs_hbm, nfa_hbm, nfb_hbm,
                  na_v, nb_v, nbrs_v, nfa_v, nfb_v):
        w = _wid()
        n0 = w * NB
        pltpu.sync_copy(na_hbm, na_v)
        pltpu.sync_copy(nb_hbm, nb_v)
        for k in range(K):
            pltpu.sync_copy(nbrs_hbm.at[pl.ds(k * NP + n0, NB)],
                            nbrs_v.at[pl.ds(k * NB, NB)])

        def nf_body(i, c):
            idx = jnp.clip(nbrs_v[pl.ds(i * 16, 16)], 0, NP - 1)
            va = plsc.load_gather(na_v, [idx])
            vb = plsc.load_gather(nb_v, [idx])
            nfa_v[pl.ds(i * 16, 16)] = plsc.bitcast(va, jnp.float32)
            nfb_v[pl.ds(i * 16, 16)] = plsc.bitcast(vb, jnp.float32)
            return c

        lax.fori_loop(0, K * GV, nf_body, 0)

        for k in range(K):
            pltpu.sync_copy(nfa_v.at[pl.ds(k * NB, NB)],
                            nfa_hbm.at[pl.ds(k * NP + n0, NB)])
            pltpu.sync_copy(nfb_v.at[pl.ds(k * NB, NB)],
                            nfb_hbm.at[pl.ds(k * NP + n0, NB)])

    return sc_count, sc_merge, sc_gather


def _leaky(x):
    return jnp.where(x >= 0, x, 0.1 * x)


def _full_spec(*s):
    return pl.BlockSpec(s, lambda: tuple(0 for _ in s))


def _tc_proj_body(x_ref, w_ref, b_ref, o_ref):
    o_ref[...] = jnp.dot(x_ref[...], w_ref[...],
                         preferred_element_type=jnp.float32) + b_ref[...]


def _tc_proj(x, w, b):
    blk = NP // 4
    return pl.pallas_call(
        _tc_proj_body,
        grid=(NP // blk,),
        in_specs=[
            pl.BlockSpec((blk, DF), lambda i: (i, 0)),
            pl.BlockSpec((DF, FD), lambda i: (0, 0)),
            pl.BlockSpec((1, FD), lambda i: (0, 0)),
        ],
        out_specs=pl.BlockSpec((blk, FD), lambda i: (i, 0)),
        out_shape=jax.ShapeDtypeStruct((NP, FD), jnp.float32),
    )(x, w, b)


def _mlp(x, w1, b1, w2, b2):
    h = _leaky(jnp.dot(x, w1, preferred_element_type=jnp.float32) + b1)
    return jnp.dot(h, w2, preferred_element_type=jnp.float32) + b2


def _tc_layer_body(e0_ref, e1_ref, e2_ref, fa_ref, fb_ref, nodes_ref, has_ref,
                   wie_ref, bie_ref, mw1_ref, mb1_ref, mw2_ref, mb2_ref,
                   uw1_ref, ub1_ref, uw2_ref, ub2_ref, o_ref):
    nodes = nodes_ref[...]
    aggr = jnp.zeros_like(nodes)
    for k, e_ref in enumerate((e0_ref, e1_ref, e2_ref)):
        ef = jnp.dot(e_ref[...], wie_ref[...],
                     preferred_element_type=jnp.float32) + bie_ref[...]
        nf = jnp.concatenate([fa_ref[...][:, k:k + 1], fb_ref[...][:, k:k + 1]],
                             axis=1)
        msg = _mlp(jnp.concatenate([ef, nf], axis=1),
                   mw1_ref[...], mb1_ref[...], mw2_ref[...], mb2_ref[...])
        aggr = aggr + msg
    new_c = _mlp(jnp.concatenate([nodes, aggr], axis=1),
                 uw1_ref[...], ub1_ref[...], uw2_ref[...], ub2_ref[...])
    mask = has_ref[...].astype(jnp.float32)
    o_ref[...] = jnp.maximum(nodes + new_c * mask, 0.0)


def _tc_layer(e_sel, nfa, nfb, nodes, has, wie, bie, mw1, mb1, mw2, mb2,
              uw1, ub1, uw2, ub2):
    return pl.pallas_call(
        _tc_layer_body,
        in_specs=[
            _full_spec(NP, DE), _full_spec(NP, DE), _full_spec(NP, DE),
            _full_spec(NP, K), _full_spec(NP, K),
            _full_spec(NP, FD), _full_spec(NP, 1),
            _full_spec(DE, FD), _full_spec(1, FD),
            _full_spec(2 * FD, HD), _full_spec(1, HD), _full_spec(HD, FD),
            _full_spec(1, FD),
            _full_spec(2 * FD, HD), _full_spec(1, HD), _full_spec(HD, FD),
            _full_spec(1, FD),
        ],
        out_specs=_full_spec(NP, FD),
        out_shape=jax.ShapeDtypeStruct((NP, FD), jnp.float32),
    )(e_sel[0], e_sel[1], e_sel[2], nfa, nfb, nodes, has,
      wie, bie, mw1, mb1, mw2, mb2, uw1, ub1, uw2, ub2)


def _tc_final_body(x_ref, w1_ref, b1_ref, w2_ref, b2_ref, o_ref):
    o_ref[...] = _mlp(x_ref[...], w1_ref[...], b1_ref[...], w2_ref[...],
                      b2_ref[...])


def _tc_final(x, w1, b1, w2, b2):
    return pl.pallas_call(
        _tc_final_body,
        in_specs=[_full_spec(NP, FD), _full_spec(FD, HD), _full_spec(1, HD),
                  _full_spec(HD, NC), _full_spec(1, NC)],
        out_specs=_full_spec(NP, NC),
        out_shape=jax.ShapeDtypeStruct((NP, NC), jnp.float32),
    )(x, w1, b1, w2, b2)


def kernel(node_feat, edge_attr, edge_index, batch,
           W_in_node, b_in_node, W_in_edge, b_in_edge,
           msg1_W1, msg1_b1, msg1_W2, msg1_b2, upd1_W1, upd1_b1, upd1_W2, upd1_b2,
           msg2_W1, msg2_b1, msg2_W2, msg2_b2, upd2_W1, upd2_b1, upd2_W2, upd2_b2,
           fin_W1, fin_b1, fin_W2, fin_b2):
    del batch
    sc_count, sc_merge, sc_gather = _sc_kernels()

    nf_pad = jnp.zeros((NP, DF), jnp.float32).at[:N].set(node_feat)
    nodes0 = _tc_proj(nf_pad, W_in_node, b_in_node.reshape(1, FD))

    src = edge_index[0]
    dst = edge_index[1]
    cnt, ceid, cdst = sc_count(src, dst)

    nodes0_i = lax.bitcast_convert_type(nodes0, jnp.int32)
    nbrs, has, esel, nf0a, nf0b = sc_merge(cnt, ceid, cdst, edge_attr,
                                           nodes0_i[:, 0], nodes0_i[:, 1])

    has2d = has.reshape(NP, 1)
    e_sel = esel.reshape(K, NP, DE)

    def planes(a, b):
        return a.reshape(K, NP).T, b.reshape(K, NP).T

    nf0at, nf0bt = planes(nf0a, nf0b)
    nodes1 = _tc_layer(e_sel, nf0at, nf0bt, nodes0, has2d,
                       W_in_edge, b_in_edge.reshape(1, FD),
                       msg1_W1, msg1_b1.reshape(1, HD), msg1_W2,
                       msg1_b2.reshape(1, FD),
                       upd1_W1, upd1_b1.reshape(1, HD), upd1_W2,
                       upd1_b2.reshape(1, FD))

    nodes1_i = lax.bitcast_convert_type(nodes1, jnp.int32)
    nf1a, nf1b = sc_gather(nodes1_i[:, 0], nodes1_i[:, 1], nbrs)
    nf1at, nf1bt = planes(nf1a, nf1b)
    nodes2 = _tc_layer(e_sel, nf1at, nf1bt, nodes1, has2d,
                       W_in_edge, b_in_edge.reshape(1, FD),
                       msg2_W1, msg2_b1.reshape(1, HD), msg2_W2,
                       msg2_b2.reshape(1, FD),
                       upd2_W1, upd2_b1.reshape(1, HD), upd2_W2,
                       upd2_b2.reshape(1, FD))

    out = _tc_final(nodes2, fin_W1, fin_b1.reshape(1, HD), fin_W2,
                    fin_b2.reshape(1, NC))
    return out[:N]
